# Initial kernel scaffold; baseline (speedup 1.0000x reference)
import jax
import jax.numpy as jnp
from jax.experimental import pallas as pl


def kernel(x, Wg, W1, W3, W2):
    raise NotImplementedError("write your pallas kernel here")


# speedup vs baseline: 3.6028x; 3.6028x over previous
import functools

import jax
import jax.numpy as jnp
from jax.experimental import pallas as pl
from jax.experimental.pallas import tpu as pltpu

E = 8
K = 2
D = 1024
DFF = 4096

BT = 256
FT = 2048
RB = 1024


def _router_body(x_ref, wg_ref, ei_ref, ew_ref):
    x = x_ref[...]
    wg = wg_ref[...]
    scores = jax.lax.dot_general(
        x, wg, (((1,), (1,)), ((), ())), preferred_element_type=jnp.float32)
    iota = jax.lax.broadcasted_iota(jnp.int32, scores.shape, 1)
    m1 = jnp.max(scores, axis=1, keepdims=True)
    i1 = jnp.min(jnp.where(scores == m1, iota, E), axis=1, keepdims=True)
    s2 = jnp.where(iota == i1, -jnp.inf, scores)
    m2 = jnp.max(s2, axis=1, keepdims=True)
    i2 = jnp.min(jnp.where(s2 == m2, iota, E), axis=1, keepdims=True)
    w1 = 1.0 / (1.0 + jnp.exp(m2 - m1))
    ei_ref[:, 0:1] = i1
    ei_ref[:, 1:2] = i2
    ew_ref[:, 0:1] = w1
    ew_ref[:, 1:2] = 1.0 - w1


def _route(xf, Wg):
    n = xf.shape[0]
    return pl.pallas_call(
        _router_body,
        grid=(n // RB,),
        in_specs=[
            pl.BlockSpec((RB, D), lambda i: (i, 0)),
            pl.BlockSpec((E, D), lambda i: (0, 0)),
        ],
        out_specs=[
            pl.BlockSpec((RB, K), lambda i: (i, 0)),
            pl.BlockSpec((RB, K), lambda i: (i, 0)),
        ],
        out_shape=[
            jax.ShapeDtypeStruct((n, K), jnp.int32),
            jax.ShapeDtypeStruct((n, K), jnp.float32),
        ],
    )(xf, Wg)


def _mlp1_body(be_ref, x_ref, w1_ref, w3_ref, h_ref):
    x = x_ref[...]
    w1 = w1_ref[0]
    w3 = w3_ref[0]
    a = jax.lax.dot_general(
        x, w1, (((1,), (1,)), ((), ())), preferred_element_type=jnp.float32)
    b = jax.lax.dot_general(
        x, w3, (((1,), (1,)), ((), ())), preferred_element_type=jnp.float32)
    h_ref[...] = a * jax.lax.logistic(a) * b


def _mlp2_body(be_ref, h_ref, w2_ref, ws_ref, y_ref):
    h = h_ref[...]
    w2 = w2_ref[0]
    out = jax.lax.dot_general(
        h, w2, (((1,), (1,)), ((), ())), preferred_element_type=jnp.float32)
    y_ref[...] = out * ws_ref[0, 0, :][:, None]


def _grouped_mlp(xs, W1, W3, W2, ws, be, nb):
    npad = nb * BT
    nf = DFF // FT
    h = pl.pallas_call(
        _mlp1_body,
        grid_spec=pltpu.PrefetchScalarGridSpec(
            num_scalar_prefetch=1,
            grid=(nf, nb),
            in_specs=[
                pl.BlockSpec((BT, D), lambda f, i, be: (i, 0)),
                pl.BlockSpec((1, FT, D), lambda f, i, be: (be[i], f, 0)),
                pl.BlockSpec((1, FT, D), lambda f, i, be: (be[i], f, 0)),
            ],
            out_specs=pl.BlockSpec((BT, FT), lambda f, i, be: (i, f)),
        ),
        out_shape=jax.ShapeDtypeStruct((npad, DFF), jnp.float32),
    )(be, xs, W1, W3)
    ws3 = ws.reshape(nb, 1, BT)
    return pl.pallas_call(
        _mlp2_body,
        grid_spec=pltpu.PrefetchScalarGridSpec(
            num_scalar_prefetch=1,
            grid=(nb,),
            in_specs=[
                pl.BlockSpec((BT, DFF), lambda i, be: (i, 0)),
                pl.BlockSpec((1, D, DFF), lambda i, be: (be[i], 0, 0)),
                pl.BlockSpec((1, 1, BT), lambda i, be: (i, 0, 0)),
            ],
            out_specs=pl.BlockSpec((BT, D), lambda i, be: (i, 0)),
        ),
        out_shape=jax.ShapeDtypeStruct((npad, D), jnp.float32),
    )(be, h, W2, ws3)


def kernel(x, Wg, W1, W3, W2):
    orig_shape = x.shape
    xf = x.reshape(-1, D)
    n = xf.shape[0]
    p = n * K
    nb = p // BT + E
    npad = nb * BT

    ei, ew = _route(xf, Wg)

    fe = ei.reshape(-1)
    oh = (fe[:, None] == jnp.arange(E, dtype=jnp.int32)[None, :]).astype(jnp.int32)
    counts = jnp.sum(oh, axis=0)
    blocks_per_e = (counts + BT - 1) // BT
    bstart = jnp.cumsum(blocks_per_e)
    padded_start = (bstart - blocks_per_e) * BT
    rank = jnp.sum(jnp.cumsum(oh, axis=0) * oh, axis=1) - 1
    dest = padded_start[fe] + rank
    tok = jnp.arange(p, dtype=jnp.int32) // K
    tok_of_slot = jnp.zeros((npad,), jnp.int32).at[dest].set(tok)
    w_sorted = jnp.zeros((npad,), jnp.float32).at[dest].set(ew.reshape(-1))
    block_expert = jnp.minimum(
        jnp.sum(jnp.arange(nb, dtype=jnp.int32)[:, None] >= bstart[None, :],
                axis=1), E - 1).astype(jnp.int32)

    xs = jnp.take(xf, tok_of_slot, axis=0)

    ys = _grouped_mlp(xs, W1, W3, W2, w_sorted, block_expert, nb)

    dest2 = dest.reshape(n, K)
    y = jnp.take(ys, dest2[:, 0], axis=0) + jnp.take(ys, dest2[:, 1], axis=0)
    return y.reshape(orig_shape)
